# Initial kernel scaffold; baseline (speedup 1.0000x reference)
import jax
import jax.numpy as jnp
from jax.experimental import pallas as pl


def kernel(x, edge_index, W1, b1, g1, be1, W2, b2, g2, be2, W3, b3, g3, be3, W4, b4, g4, be4, W5, b5, g5, be5, fcW, fcb):
    raise NotImplementedError("write your pallas kernel here")


# speedup vs baseline: 9.3034x; 9.3034x over previous
import functools

import jax
import jax.numpy as jnp
from jax import lax
from jax.experimental import pallas as pl
from jax.experimental.pallas import tpu as pltpu
from jax.experimental.pallas import tpu_sc as plsc

N = 100000
E = 1600000
NC = 2
NS = 16
CH = 100
ER = E // CH
KS = 8
RPT = ER // NS
NSTEP = RPT // KS
RB = 1000
NBLK = N // RB
B = 4000
NB = N // B
F32 = jnp.float32

_mesh = plsc.VectorSubcoreMesh(core_axis_name="c", subcore_axis_name="s")


@functools.partial(
    pl.kernel,
    out_type=jax.ShapeDtypeStruct((NC, N, 16), F32),
    mesh=_mesh,
    compiler_params=pltpu.CompilerParams(use_tc_tiling_on_sc=False),
    scratch_types=[
        pltpu.VMEM((KS, CH), jnp.int32),
        pltpu.VMEM((CH, 16), F32),
        pltpu.VMEM((RB, 16), F32),
        pltpu.VMEM_SHARED((N, 16), F32),
        pltpu.SemaphoreType.DMA,
    ],
)
def _deg_kernel(src2_hbm, dst2_hbm, out_hbm, idx, ones_r, zbuf, acc, sem):
    c = lax.axis_index("c")
    s = lax.axis_index("s")
    one = jnp.ones((16,), F32)
    zero = jnp.zeros((16,), F32)

    @pl.loop(0, CH)
    def _(i):
        ones_r[i, :] = one

    @pl.loop(0, RB)
    def _(i):
        zbuf[i, :] = zero

    @pl.loop(s, NBLK, step=NS)
    def _(j):
        pltpu.sync_copy(zbuf, acc.at[pl.ds(j * RB, RB)])

    plsc.subcore_barrier()

    row0 = s * RPT

    def count_pass(e2_hbm):
        @pl.loop(0, NSTEP)
        def _(j):
            r = pl.multiple_of(row0 + j * KS, 8)
            pltpu.sync_copy(e2_hbm.at[pl.ds(r, KS)], idx)
            descs = [pltpu.async_copy(ones_r, acc.at[idx.at[k]], sem,
                                      add=True) for k in range(KS)]
            for d in descs:
                d.wait()

    @pl.when(c == 0)
    def _():
        count_pass(src2_hbm)

    @pl.when(c == 1)
    def _():
        count_pass(dst2_hbm)

    plsc.subcore_barrier()

    @pl.loop(s, NBLK, step=NS)
    def _(j):
        pltpu.sync_copy(acc.at[pl.ds(j * RB, RB)],
                        out_hbm.at[c, pl.ds(j * RB, RB)])


@functools.partial(
    pl.kernel,
    out_type=[jax.ShapeDtypeStruct((N, 16), F32),
              jax.ShapeDtypeStruct((N, 16), F32)],
    mesh=_mesh,
    compiler_params=pltpu.CompilerParams(use_tc_tiling_on_sc=False),
    scratch_types=[
        pltpu.VMEM((KS, CH), jnp.int32),
        pltpu.VMEM((KS, CH), jnp.int32),
        pltpu.VMEM((KS, CH, 16), F32),
        pltpu.VMEM((RB, 16), F32),
        pltpu.VMEM_SHARED((N, 16), F32),
        pltpu.SemaphoreType.DMA,
        pltpu.SemaphoreType.DMA,
    ],
)
def _spmm_kernel(hlo_hbm, hhi_hbm, src2_hbm, dst2_hbm, olo_hbm, ohi_hbm,
                 sidx, didx, rows, zbuf, acc, gsem, ssem):
    c = lax.axis_index("c")
    s = lax.axis_index("s")
    zero = jnp.zeros((16,), F32)

    @pl.loop(0, RB)
    def _(i):
        zbuf[i, :] = zero

    @pl.loop(s, NBLK, step=NS)
    def _(j):
        pltpu.sync_copy(zbuf, acc.at[pl.ds(j * RB, RB)])

    plsc.subcore_barrier()

    row0 = s * RPT

    def edge_pass(h_hbm):
        @pl.loop(0, NSTEP)
        def _(j):
            r = pl.multiple_of(row0 + j * KS, 8)
            pltpu.sync_copy(src2_hbm.at[pl.ds(r, KS)], sidx)
            pltpu.sync_copy(dst2_hbm.at[pl.ds(r, KS)], didx)
            gd = [pltpu.async_copy(h_hbm.at[sidx.at[k]], rows.at[k], gsem)
                  for k in range(KS)]
            for d in gd:
                d.wait()
            sd = [pltpu.async_copy(rows.at[k], acc.at[didx.at[k]], ssem,
                                   add=True) for k in range(KS)]
            for d in sd:
                d.wait()

    @pl.when(c == 0)
    def _():
        edge_pass(hlo_hbm)

    @pl.when(c == 1)
    def _():
        edge_pass(hhi_hbm)

    plsc.subcore_barrier()

    def copy_out(o_hbm):
        @pl.loop(s, NBLK, step=NS)
        def _(j):
            pltpu.sync_copy(acc.at[pl.ds(j * RB, RB)],
                            o_hbm.at[pl.ds(j * RB, RB)])

    @pl.when(c == 0)
    def _():
        copy_out(olo_hbm)

    @pl.when(c == 1)
    def _():
        copy_out(ohi_hbm)


def _pre1_body(x_ref, cnt_ref, w_ref, lo_ref, hi_ref, si_ref, so_ref):
    so = lax.rsqrt(jnp.maximum(cnt_ref[0, :, 0:1], 1.0))
    si = lax.rsqrt(jnp.maximum(cnt_ref[1, :, 0:1], 1.0))
    z = jnp.dot(x_ref[...] * so, w_ref[...], preferred_element_type=F32)
    lo_ref[...] = z[:, :16]
    hi_ref[...] = z[:, 16:]
    si_ref[...] = si
    so_ref[...] = so


def _pre1(x, cnt, W1):
    return pl.pallas_call(
        _pre1_body,
        grid=(NB,),
        in_specs=[
            pl.BlockSpec((B, 33), lambda i: (i, 0)),
            pl.BlockSpec((NC, B, 16), lambda i: (0, i, 0)),
            pl.BlockSpec((33, 32), lambda i: (0, 0)),
        ],
        out_specs=[
            pl.BlockSpec((B, 16), lambda i: (i, 0)),
            pl.BlockSpec((B, 16), lambda i: (i, 0)),
            pl.BlockSpec((B, 1), lambda i: (i, 0)),
            pl.BlockSpec((B, 1), lambda i: (i, 0)),
        ],
        out_shape=[
            jax.ShapeDtypeStruct((N, 16), F32),
            jax.ShapeDtypeStruct((N, 16), F32),
            jax.ShapeDtypeStruct((N, 1), F32),
            jax.ShapeDtypeStruct((N, 1), F32),
        ],
    )(x, cnt, W1)


def _stats_body(lo_ref, hi_ref, si_ref, b_ref, st_ref):
    t = jnp.concatenate([lo_ref[...], hi_ref[...]], axis=1)
    t = t * si_ref[...] + b_ref[...]
    s1 = jnp.sum(t, axis=0)
    s2 = jnp.sum(t * t, axis=0)
    upd = jnp.concatenate([s1[None], s2[None], jnp.zeros((6, 32), F32)],
                          axis=0)

    @pl.when(pl.program_id(0) == 0)
    def _():
        st_ref[...] = jnp.zeros_like(st_ref)

    st_ref[...] += upd


def _stats(alo, ahi, si, b):
    return pl.pallas_call(
        _stats_body,
        grid=(NB,),
        in_specs=[
            pl.BlockSpec((B, 16), lambda i: (i, 0)),
            pl.BlockSpec((B, 16), lambda i: (i, 0)),
            pl.BlockSpec((B, 1), lambda i: (i, 0)),
            pl.BlockSpec((1, 32), lambda i: (0, 0)),
        ],
        out_specs=pl.BlockSpec((8, 32), lambda i: (0, 0)),
        out_shape=jax.ShapeDtypeStruct((8, 32), F32),
    )(alo, ahi, si, b)


def _bn_lrelu(t, st_ref, g_ref, be_ref):
    mu = st_ref[0, :] * (1.0 / N)
    var = st_ref[1, :] * (1.0 / N) - mu * mu
    y = (t - mu[None, :]) * lax.rsqrt(var + 1e-5)[None, :]
    y = y * g_ref[...] + be_ref[...]
    return jnp.where(y >= 0, y, 0.01 * y)


def _mid_body(lo_ref, hi_ref, si_ref, so_ref, st_ref, b_ref, g_ref, be_ref,
              w_ref, olo_ref, ohi_ref):
    t = jnp.concatenate([lo_ref[...], hi_ref[...]], axis=1)
    t = t * si_ref[...] + b_ref[...]
    y = _bn_lrelu(t, st_ref, g_ref, be_ref)
    z = jnp.dot(y * so_ref[...], w_ref[...], preferred_element_type=F32)
    olo_ref[...] = z[:, :16]
    ohi_ref[...] = z[:, 16:]


def _mid(alo, ahi, si, so, st, b, g, be, Wn):
    return pl.pallas_call(
        _mid_body,
        grid=(NB,),
        in_specs=[
            pl.BlockSpec((B, 16), lambda i: (i, 0)),
            pl.BlockSpec((B, 16), lambda i: (i, 0)),
            pl.BlockSpec((B, 1), lambda i: (i, 0)),
            pl.BlockSpec((B, 1), lambda i: (i, 0)),
            pl.BlockSpec((8, 32), lambda i: (0, 0)),
            pl.BlockSpec((1, 32), lambda i: (0, 0)),
            pl.BlockSpec((1, 32), lambda i: (0, 0)),
            pl.BlockSpec((1, 32), lambda i: (0, 0)),
            pl.BlockSpec((32, 32), lambda i: (0, 0)),
        ],
        out_specs=[
            pl.BlockSpec((B, 16), lambda i: (i, 0)),
            pl.BlockSpec((B, 16), lambda i: (i, 0)),
        ],
        out_shape=[
            jax.ShapeDtypeStruct((N, 16), F32),
            jax.ShapeDtypeStruct((N, 16), F32),
        ],
    )(alo, ahi, si, so, st, b, g, be, Wn)


def _fin_body(lo_ref, hi_ref, si_ref, st_ref, b_ref, g_ref, be_ref, w_ref,
              fb_ref, o_ref):
    t = jnp.concatenate([lo_ref[...], hi_ref[...]], axis=1)
    t = t * si_ref[...] + b_ref[...]
    y = _bn_lrelu(t, st_ref, g_ref, be_ref)
    o_ref[...] = jnp.dot(y, w_ref[...], preferred_element_type=F32) + fb_ref[...]


def _fin(alo, ahi, si, st, b, g, be, fcW, fcb):
    return pl.pallas_call(
        _fin_body,
        grid=(NB,),
        in_specs=[
            pl.BlockSpec((B, 16), lambda i: (i, 0)),
            pl.BlockSpec((B, 16), lambda i: (i, 0)),
            pl.BlockSpec((B, 1), lambda i: (i, 0)),
            pl.BlockSpec((8, 32), lambda i: (0, 0)),
            pl.BlockSpec((1, 32), lambda i: (0, 0)),
            pl.BlockSpec((1, 32), lambda i: (0, 0)),
            pl.BlockSpec((1, 32), lambda i: (0, 0)),
            pl.BlockSpec((32, 2), lambda i: (0, 0)),
            pl.BlockSpec((1, 2), lambda i: (0, 0)),
        ],
        out_specs=pl.BlockSpec((B, 2), lambda i: (i, 0)),
        out_shape=jax.ShapeDtypeStruct((N, 2), F32),
    )(alo, ahi, si, st, b, g, be, fcW, fcb)


def kernel(x, edge_index, W1, b1, g1, be1, W2, b2, g2, be2, W3, b3, g3, be3,
           W4, b4, g4, be4, W5, b5, g5, be5, fcW, fcb):
    src2 = edge_index[0].reshape(ER, CH)
    dst2 = edge_index[1].reshape(ER, CH)
    cnt = _deg_kernel(src2, dst2)
    lo, hi, si, so = _pre1(x, cnt, W1)

    bs = [b1, b2, b3, b4, b5]
    gs = [g1, g2, g3, g4, g5]
    bes = [be1, be2, be3, be4, be5]
    Wn = [W2, W3, W4, W5]

    for i in range(5):
        alo, ahi = _spmm_kernel(lo, hi, src2, dst2)
        b2d = bs[i].reshape(1, 32)
        g2d = gs[i].reshape(1, 32)
        be2d = bes[i].reshape(1, 32)
        st = _stats(alo, ahi, si, b2d)
        if i < 4:
            lo, hi = _mid(alo, ahi, si, so, st, b2d, g2d, be2d, Wn[i])
        else:
            out = _fin(alo, ahi, si, st, b2d, g2d, be2d, fcW,
                       fcb.reshape(1, 2))
    return out


# speedup vs baseline: 12.0460x; 1.2948x over previous
import functools

import jax
import jax.numpy as jnp
from jax import lax
from jax.experimental import pallas as pl
from jax.experimental.pallas import tpu as pltpu
from jax.experimental.pallas import tpu_sc as plsc

N = 100000
E = 1600000
NC = 2
NS = 16
CH = 100
ER = E // CH
KS = 8
RPT = ER // NS
NSTEP = RPT // KS
RB = 1000
NBLK = N // RB
ZB = 128
NZB = N // ZB
ZTAIL = N - NZB * ZB
B = 4000
NB = N // B
F32 = jnp.float32

_mesh = plsc.VectorSubcoreMesh(core_axis_name="c", subcore_axis_name="s")


def _zero_acc(zbuf, acc, s):
    zero = jnp.zeros((16,), F32)

    @pl.loop(0, ZB)
    def _(i):
        zbuf[i, :] = zero

    @pl.loop(s, NZB, step=NS)
    def _(j):
        pltpu.sync_copy(zbuf, acc.at[pl.ds(j * ZB, ZB)])

    @pl.when(s == 0)
    def _():
        pltpu.sync_copy(zbuf.at[pl.ds(0, ZTAIL)],
                        acc.at[pl.ds(NZB * ZB, ZTAIL)])


@functools.partial(
    pl.kernel,
    out_type=jax.ShapeDtypeStruct((NC, N, 16), F32),
    mesh=_mesh,
    compiler_params=pltpu.CompilerParams(use_tc_tiling_on_sc=False),
    scratch_types=[
        pltpu.VMEM((KS, CH), jnp.int32),
        pltpu.VMEM((CH, 16), F32),
        pltpu.VMEM((ZB, 16), F32),
        pltpu.VMEM_SHARED((N, 16), F32),
        pltpu.SemaphoreType.DMA,
    ],
)
def _deg_kernel(src2_hbm, dst2_hbm, out_hbm, idx, ones_r, zbuf, acc, sem):
    c = lax.axis_index("c")
    s = lax.axis_index("s")
    one = jnp.ones((16,), F32)

    @pl.loop(0, CH)
    def _(i):
        ones_r[i, :] = one

    _zero_acc(zbuf, acc, s)

    plsc.subcore_barrier()

    row0 = s * RPT

    def count_pass(e2_hbm):
        @pl.loop(0, NSTEP)
        def _(j):
            r = pl.multiple_of(row0 + j * KS, 8)
            pltpu.sync_copy(e2_hbm.at[pl.ds(r, KS)], idx)
            descs = [pltpu.async_copy(ones_r, acc.at[idx.at[k]], sem,
                                      add=True) for k in range(KS)]
            for d in descs:
                d.wait()

    @pl.when(c == 0)
    def _():
        count_pass(src2_hbm)

    @pl.when(c == 1)
    def _():
        count_pass(dst2_hbm)

    plsc.subcore_barrier()

    @pl.loop(s, NBLK, step=NS)
    def _(j):
        pltpu.sync_copy(acc.at[pl.ds(j * RB, RB)],
                        out_hbm.at[c, pl.ds(j * RB, RB)])


@functools.partial(
    pl.kernel,
    out_type=[jax.ShapeDtypeStruct((N, 16), F32),
              jax.ShapeDtypeStruct((N, 16), F32)],
    mesh=_mesh,
    compiler_params=pltpu.CompilerParams(use_tc_tiling_on_sc=False),
    scratch_types=[
        pltpu.VMEM((KS, CH), jnp.int32),
        pltpu.VMEM((KS, CH), jnp.int32),
        pltpu.VMEM((KS, CH), jnp.int32),
        pltpu.VMEM((KS, CH), jnp.int32),
        pltpu.VMEM((KS * CH, 16), F32),
        pltpu.VMEM((KS * CH, 16), F32),
        pltpu.VMEM((ZB, 16), F32),
        pltpu.VMEM_SHARED((N, 16), F32),
        pltpu.SemaphoreType.DMA,
        pltpu.SemaphoreType.DMA,
        pltpu.SemaphoreType.DMA,
        pltpu.SemaphoreType.DMA,
    ],
)
def _spmm_kernel(hlo_hbm, hhi_hbm, src2_hbm, dst2_hbm, olo_hbm, ohi_hbm,
                 sidx0, didx0, sidx1, didx1, rows0, rows1, zbuf, acc,
                 gsem0, gsem1, ssem0, ssem1):
    c = lax.axis_index("c")
    s = lax.axis_index("s")

    _zero_acc(zbuf, acc, s)

    plsc.subcore_barrier()

    row0 = s * RPT
    HB = KS * CH

    def edge_pass(h_hbm):
        def load_idx(j, si, di):
            r = pl.multiple_of(row0 + j * KS, 8)
            pltpu.sync_copy(src2_hbm.at[pl.ds(r, KS)], si)
            pltpu.sync_copy(dst2_hbm.at[pl.ds(r, KS)], di)

        def fire_g(si, rb, sem):
            for k in range(KS):
                pltpu.async_copy(h_hbm.at[si.at[k]],
                                 rb.at[pl.ds(k * CH, CH)], sem)

        def fire_s(di, rb, sem):
            for k in range(KS):
                pltpu.async_copy(rb.at[pl.ds(k * CH, CH)],
                                 acc.at[di.at[k]], sem, add=True)

        def drain(sem, rb):
            pltpu.make_async_copy(h_hbm.at[pl.ds(0, HB)], rb, sem).wait()

        load_idx(0, sidx0, didx0)
        fire_g(sidx0, rows0, gsem0)

        @pl.loop(0, NSTEP // 2)
        def _(t):
            j = 2 * t
            load_idx(j + 1, sidx1, didx1)
            fire_g(sidx1, rows1, gsem1)
            drain(gsem0, rows0)
            fire_s(didx0, rows0, ssem0)
            drain(ssem0, rows0)

            @pl.when(t + 1 < NSTEP // 2)
            def _():
                load_idx(j + 2, sidx0, didx0)
                fire_g(sidx0, rows0, gsem0)

            drain(gsem1, rows1)
            fire_s(didx1, rows1, ssem1)
            drain(ssem1, rows1)

        if NSTEP % 2 == 1:
            load_idx(NSTEP - 1, sidx0, didx0)
            fire_g(sidx0, rows0, gsem0)
            drain(gsem0, rows0)
            fire_s(didx0, rows0, ssem0)
            drain(ssem0, rows0)

    @pl.when(c == 0)
    def _():
        edge_pass(hlo_hbm)

    @pl.when(c == 1)
    def _():
        edge_pass(hhi_hbm)

    plsc.subcore_barrier()

    def copy_out(o_hbm):
        @pl.loop(s, NBLK, step=NS)
        def _(j):
            pltpu.sync_copy(acc.at[pl.ds(j * RB, RB)],
                            o_hbm.at[pl.ds(j * RB, RB)])

    @pl.when(c == 0)
    def _():
        copy_out(olo_hbm)

    @pl.when(c == 1)
    def _():
        copy_out(ohi_hbm)


def _pre1_body(x_ref, cnt_ref, w_ref, lo_ref, hi_ref, si_ref, so_ref):
    so = lax.rsqrt(jnp.maximum(cnt_ref[0, :, 0:1], 1.0))
    si = lax.rsqrt(jnp.maximum(cnt_ref[1, :, 0:1], 1.0))
    z = jnp.dot(x_ref[...] * so, w_ref[...], preferred_element_type=F32)
    lo_ref[...] = z[:, :16]
    hi_ref[...] = z[:, 16:]
    si_ref[...] = si
    so_ref[...] = so


def _pre1(x, cnt, W1):
    return pl.pallas_call(
        _pre1_body,
        grid=(NB,),
        in_specs=[
            pl.BlockSpec((B, 33), lambda i: (i, 0)),
            pl.BlockSpec((NC, B, 16), lambda i: (0, i, 0)),
            pl.BlockSpec((33, 32), lambda i: (0, 0)),
        ],
        out_specs=[
            pl.BlockSpec((B, 16), lambda i: (i, 0)),
            pl.BlockSpec((B, 16), lambda i: (i, 0)),
            pl.BlockSpec((B, 1), lambda i: (i, 0)),
            pl.BlockSpec((B, 1), lambda i: (i, 0)),
        ],
        out_shape=[
            jax.ShapeDtypeStruct((N, 16), F32),
            jax.ShapeDtypeStruct((N, 16), F32),
            jax.ShapeDtypeStruct((N, 1), F32),
            jax.ShapeDtypeStruct((N, 1), F32),
        ],
    )(x, cnt, W1)


def _stats_body(lo_ref, hi_ref, si_ref, b_ref, st_ref):
    t = jnp.concatenate([lo_ref[...], hi_ref[...]], axis=1)
    t = t * si_ref[...] + b_ref[...]
    s1 = jnp.sum(t, axis=0)
    s2 = jnp.sum(t * t, axis=0)
    upd = jnp.concatenate([s1[None], s2[None], jnp.zeros((6, 32), F32)],
                          axis=0)

    @pl.when(pl.program_id(0) == 0)
    def _():
        st_ref[...] = jnp.zeros_like(st_ref)

    st_ref[...] += upd


def _stats(alo, ahi, si, b):
    return pl.pallas_call(
        _stats_body,
        grid=(NB,),
        in_specs=[
            pl.BlockSpec((B, 16), lambda i: (i, 0)),
            pl.BlockSpec((B, 16), lambda i: (i, 0)),
            pl.BlockSpec((B, 1), lambda i: (i, 0)),
            pl.BlockSpec((1, 32), lambda i: (0, 0)),
        ],
        out_specs=pl.BlockSpec((8, 32), lambda i: (0, 0)),
        out_shape=jax.ShapeDtypeStruct((8, 32), F32),
    )(alo, ahi, si, b)


def _bn_lrelu(t, st_ref, g_ref, be_ref):
    mu = st_ref[0, :] * (1.0 / N)
    var = st_ref[1, :] * (1.0 / N) - mu * mu
    y = (t - mu[None, :]) * lax.rsqrt(var + 1e-5)[None, :]
    y = y * g_ref[...] + be_ref[...]
    return jnp.where(y >= 0, y, 0.01 * y)


def _mid_body(lo_ref, hi_ref, si_ref, so_ref, st_ref, b_ref, g_ref, be_ref,
              w_ref, olo_ref, ohi_ref):
    t = jnp.concatenate([lo_ref[...], hi_ref[...]], axis=1)
    t = t * si_ref[...] + b_ref[...]
    y = _bn_lrelu(t, st_ref, g_ref, be_ref)
    z = jnp.dot(y * so_ref[...], w_ref[...], preferred_element_type=F32)
    olo_ref[...] = z[:, :16]
    ohi_ref[...] = z[:, 16:]


def _mid(alo, ahi, si, so, st, b, g, be, Wn):
    return pl.pallas_call(
        _mid_body,
        grid=(NB,),
        in_specs=[
            pl.BlockSpec((B, 16), lambda i: (i, 0)),
            pl.BlockSpec((B, 16), lambda i: (i, 0)),
            pl.BlockSpec((B, 1), lambda i: (i, 0)),
            pl.BlockSpec((B, 1), lambda i: (i, 0)),
            pl.BlockSpec((8, 32), lambda i: (0, 0)),
            pl.BlockSpec((1, 32), lambda i: (0, 0)),
            pl.BlockSpec((1, 32), lambda i: (0, 0)),
            pl.BlockSpec((1, 32), lambda i: (0, 0)),
            pl.BlockSpec((32, 32), lambda i: (0, 0)),
        ],
        out_specs=[
            pl.BlockSpec((B, 16), lambda i: (i, 0)),
            pl.BlockSpec((B, 16), lambda i: (i, 0)),
        ],
        out_shape=[
            jax.ShapeDtypeStruct((N, 16), F32),
            jax.ShapeDtypeStruct((N, 16), F32),
        ],
    )(alo, ahi, si, so, st, b, g, be, Wn)


def _fin_body(lo_ref, hi_ref, si_ref, st_ref, b_ref, g_ref, be_ref, w_ref,
              fb_ref, o_ref):
    t = jnp.concatenate([lo_ref[...], hi_ref[...]], axis=1)
    t = t * si_ref[...] + b_ref[...]
    y = _bn_lrelu(t, st_ref, g_ref, be_ref)
    o_ref[...] = jnp.dot(y, w_ref[...], preferred_element_type=F32) + fb_ref[...]


def _fin(alo, ahi, si, st, b, g, be, fcW, fcb):
    return pl.pallas_call(
        _fin_body,
        grid=(NB,),
        in_specs=[
            pl.BlockSpec((B, 16), lambda i: (i, 0)),
            pl.BlockSpec((B, 16), lambda i: (i, 0)),
            pl.BlockSpec((B, 1), lambda i: (i, 0)),
            pl.BlockSpec((8, 32), lambda i: (0, 0)),
            pl.BlockSpec((1, 32), lambda i: (0, 0)),
            pl.BlockSpec((1, 32), lambda i: (0, 0)),
            pl.BlockSpec((1, 32), lambda i: (0, 0)),
            pl.BlockSpec((32, 2), lambda i: (0, 0)),
            pl.BlockSpec((1, 2), lambda i: (0, 0)),
        ],
        out_specs=pl.BlockSpec((B, 2), lambda i: (i, 0)),
        out_shape=jax.ShapeDtypeStruct((N, 2), F32),
    )(alo, ahi, si, st, b, g, be, fcW, fcb)


def kernel(x, edge_index, W1, b1, g1, be1, W2, b2, g2, be2, W3, b3, g3, be3,
           W4, b4, g4, be4, W5, b5, g5, be5, fcW, fcb):
    src2 = edge_index[0].reshape(ER, CH)
    dst2 = edge_index[1].reshape(ER, CH)
    cnt = _deg_kernel(src2, dst2)
    lo, hi, si, so = _pre1(x, cnt, W1)

    bs = [b1, b2, b3, b4, b5]
    gs = [g1, g2, g3, g4, g5]
    bes = [be1, be2, be3, be4, be5]
    Wn = [W2, W3, W4, W5]

    for i in range(5):
        alo, ahi = _spmm_kernel(lo, hi, src2, dst2)
        b2d = bs[i].reshape(1, 32)
        g2d = gs[i].reshape(1, 32)
        be2d = bes[i].reshape(1, 32)
        st = _stats(alo, ahi, si, b2d)
        if i < 4:
            lo, hi = _mid(alo, ahi, si, so, st, b2d, g2d, be2d, Wn[i])
        else:
            out = _fin(alo, ahi, si, st, b2d, g2d, be2d, fcW,
                       fcb.reshape(1, 2))
    return out
